# baseline (speedup 1.0000x reference)
import functools

import jax
import jax.numpy as jnp
from jax import lax
from jax.experimental import pallas as pl
from jax.experimental.pallas import tpu as pltpu
from jax.experimental.pallas import tpu_sc as plsc

ATOM = 128
EDGE_F = 16
OUT_F = 128
N_NODES = 10000
N_EDGES = 320000

NC = 2
NS = 16
NW = NC * NS
EPW = N_EDGES // NW
CH = 80
NCHUNK = EPW // CH


def _project_body(h_ref, ws_ref, wt_ref, a_ref, b_ref):
    h = h_ref[...]
    a_ref[...] = jnp.dot(h, ws_ref[...], preferred_element_type=jnp.float32)
    b_ref[...] = jnp.dot(h, wt_ref[...], preferred_element_type=jnp.float32)


def _project(h, ws, wt):
    return pl.pallas_call(
        _project_body,
        out_shape=(
            jax.ShapeDtypeStruct((N_NODES, OUT_F), jnp.float32),
            jax.ShapeDtypeStruct((N_NODES, OUT_F), jnp.float32),
        ),
    )(h, ws, wt)


NB = 5
NBT = 3
LEAD = 2


def _gather_add_body(a_hbm, b_hbm, src_hbm, dst_hbm, g_hbm,
                     idx_s, idx_t, bufs_s, bufs_t, *sems):
    gsem = sems[:NB]
    wsem = sems[NB:]
    wid = lax.axis_index("s") * NC + lax.axis_index("c")
    base0 = wid * EPW

    pltpu.sync_copy(src_hbm.at[wid], idx_s)
    pltpu.sync_copy(dst_hbm.at[wid], idx_t)

    def fire(chunk, sb):
        pltpu.async_copy(a_hbm.at[idx_s.at[chunk]], bufs_s.at[sb], gsem[sb])
        pltpu.async_copy(
            b_hbm.at[idx_t.at[chunk]], bufs_t.at[chunk % NBT], gsem[sb]
        )

    for p in range(LEAD):
        fire(p, p)

    def outer(i, _):
        c0 = i * NB
        for b in range(NB):
            chunk = c0 + b
            sb = (b + LEAD) % NB
            pre = chunk + LEAD

            @pl.when(pre < NCHUNK)
            def _():
                @pl.when(pre >= NB)
                def _():
                    pltpu.make_async_copy(
                        bufs_s.at[sb],
                        g_hbm.at[pl.ds(base0 + (pre - NB) * CH, CH)],
                        wsem[sb],
                    ).wait()

                fire(pre, sb)

            tslot = chunk % NBT
            pltpu.make_async_copy(
                a_hbm.at[idx_s.at[chunk]], bufs_s.at[b], gsem[b]
            ).wait()
            pltpu.make_async_copy(
                b_hbm.at[idx_t.at[chunk]], bufs_t.at[tslot], gsem[b]
            ).wait()

            @plsc.parallel_loop(0, CH, 1, unroll=4)
            def _(j):
                for k in range(OUT_F // 16):
                    sl = pl.ds(k * 16, 16)
                    bufs_s[b, j, sl] = bufs_s[b, j, sl] + bufs_t[tslot, j, sl]
            pltpu.async_copy(
                bufs_s.at[b], g_hbm.at[pl.ds(base0 + chunk * CH, CH)], wsem[b]
            )
        return 0

    lax.fori_loop(0, NCHUNK // NB, outer, 0)

    for b in range(NB):
        pltpu.make_async_copy(
            bufs_s.at[b],
            g_hbm.at[pl.ds(base0 + (NCHUNK - NB + b) * CH, CH)],
            wsem[b],
        ).wait()


def _gather_add(a, b, src3, dst3):
    mesh = plsc.VectorSubcoreMesh(
        core_axis_name="c", subcore_axis_name="s", num_cores=NC, num_subcores=NS
    )
    return pl.kernel(
        _gather_add_body,
        out_type=jax.ShapeDtypeStruct((N_EDGES, OUT_F), jnp.float32),
        mesh=mesh,
        scratch_types=[
            pltpu.VMEM((NCHUNK, CH), jnp.int32),
            pltpu.VMEM((NCHUNK, CH), jnp.int32),
            pltpu.VMEM((NB, CH, OUT_F), jnp.float32),
            pltpu.VMEM((NBT, CH, OUT_F), jnp.float32),
        ]
        + [pltpu.SemaphoreType.DMA] * (2 * NB),
    )(a, b, src3, dst3)


EB = 2000


def _combine_body(g_ref, m_ref, wm_ref, o_ref):
    x = g_ref[...] + jnp.dot(
        m_ref[...], wm_ref[...], preferred_element_type=jnp.float32
    )
    o_ref[...] = x * jax.nn.sigmoid(x)


def _combine(g, m, wm):
    grid = (N_EDGES // EB,)
    return pl.pallas_call(
        _combine_body,
        grid=grid,
        in_specs=[
            pl.BlockSpec((EB, OUT_F), lambda i: (i, 0)),
            pl.BlockSpec((EB, EDGE_F), lambda i: (i, 0)),
            pl.BlockSpec((EDGE_F, OUT_F), lambda i: (0, 0)),
        ],
        out_specs=pl.BlockSpec((EB, OUT_F), lambda i: (i, 0)),
        out_shape=jax.ShapeDtypeStruct((N_EDGES, OUT_F), jnp.float32),
    )(g, m, wm)


@jax.jit
def kernel(h, m, edge_index, W):
    ws = W[:ATOM]
    wt = W[ATOM : 2 * ATOM]
    wm = W[2 * ATOM :]
    src = edge_index[0].astype(jnp.int32).reshape(NW, NCHUNK, CH)
    dst = edge_index[1].astype(jnp.int32).reshape(NW, NCHUNK, CH)

    a, b = _project(h, ws, wt)
    g = _gather_add(a, b, src, dst)
    return _combine(g, m, wm)


# speedup vs baseline: 1.1302x; 1.1302x over previous
import functools

import jax
import jax.numpy as jnp
from jax import lax
from jax.experimental import pallas as pl
from jax.experimental.pallas import tpu as pltpu
from jax.experimental.pallas import tpu_sc as plsc

ATOM = 128
EDGE_F = 16
OUT_F = 128
N_NODES = 10000
N_EDGES = 320000

NC = 2
NS = 16
NW = NC * NS
EPW = N_EDGES // NW
CH = 80
NCHUNK = EPW // CH


def _project_body(h_ref, ws_ref, wt_ref, a_ref, b_ref):
    h = h_ref[...]
    a_ref[...] = jnp.dot(h, ws_ref[...], preferred_element_type=jnp.float32)
    b_ref[...] = jnp.dot(h, wt_ref[...], preferred_element_type=jnp.float32)


def _project(h, ws, wt):
    return pl.pallas_call(
        _project_body,
        out_shape=(
            jax.ShapeDtypeStruct((N_NODES, OUT_F), jnp.float32),
            jax.ShapeDtypeStruct((N_NODES, OUT_F), jnp.float32),
        ),
    )(h, ws, wt)


NB = 5
NBT = 3
LEAD = 2


def _gather_add_body(a_hbm, b_hbm, src_hbm, dst_hbm, g_hbm,
                     idx_s, idx_t, bufs_s, bufs_t, *sems):
    gsem = sems[:NB]
    wsem = sems[NB:]
    wid = lax.axis_index("s") * NC + lax.axis_index("c")
    base0 = wid * EPW

    pltpu.sync_copy(src_hbm.at[wid], idx_s)
    pltpu.sync_copy(dst_hbm.at[wid], idx_t)

    def fire(chunk, sb):
        pltpu.async_copy(a_hbm.at[idx_s.at[chunk]], bufs_s.at[sb], gsem[sb])
        pltpu.async_copy(
            b_hbm.at[idx_t.at[chunk]], bufs_t.at[chunk % NBT], gsem[sb]
        )

    for p in range(LEAD):
        fire(p, p)

    def outer(i, _):
        c0 = i * NB
        for b in range(NB):
            chunk = c0 + b
            sb = (b + LEAD) % NB
            pre = chunk + LEAD

            @pl.when(pre < NCHUNK)
            def _():
                @pl.when(pre >= NB)
                def _():
                    pltpu.make_async_copy(
                        bufs_s.at[sb],
                        g_hbm.at[pl.ds(base0 + (pre - NB) * CH, CH)],
                        wsem[sb],
                    ).wait()

                fire(pre, sb)

            tslot = chunk % NBT
            pltpu.make_async_copy(
                a_hbm.at[idx_s.at[chunk]], bufs_s.at[b], gsem[b]
            ).wait()
            pltpu.make_async_copy(
                b_hbm.at[idx_t.at[chunk]], bufs_t.at[tslot], gsem[b]
            ).wait()

            @plsc.parallel_loop(0, CH, 1, unroll=4)
            def _(j):
                for k in range(OUT_F // 16):
                    sl = pl.ds(k * 16, 16)
                    bufs_s[b, j, sl] = bufs_s[b, j, sl] + bufs_t[tslot, j, sl]
            pltpu.async_copy(
                bufs_s.at[b], g_hbm.at[pl.ds(base0 + chunk * CH, CH)], wsem[b]
            )
        return 0

    lax.fori_loop(0, NCHUNK // NB, outer, 0)

    for b in range(NB):
        pltpu.make_async_copy(
            bufs_s.at[b],
            g_hbm.at[pl.ds(base0 + (NCHUNK - NB + b) * CH, CH)],
            wsem[b],
        ).wait()


def _gather_add(a, b, src3, dst3):
    mesh = plsc.VectorSubcoreMesh(
        core_axis_name="c", subcore_axis_name="s", num_cores=NC, num_subcores=NS
    )
    return pl.kernel(
        _gather_add_body,
        out_type=jax.ShapeDtypeStruct((N_EDGES, OUT_F), jnp.float32),
        mesh=mesh,
        scratch_types=[
            pltpu.VMEM((NCHUNK, CH), jnp.int32),
            pltpu.VMEM((NCHUNK, CH), jnp.int32),
            pltpu.VMEM((NB, CH, OUT_F), jnp.float32),
            pltpu.VMEM((NBT, CH, OUT_F), jnp.float32),
        ]
        + [pltpu.SemaphoreType.DMA] * (2 * NB),
    )(a, b, src3, dst3)


EB = 8000


def _combine_body(g_ref, m_ref, wm_ref, o_ref):
    x = g_ref[...] + jnp.dot(
        m_ref[...], wm_ref[...], preferred_element_type=jnp.float32
    )
    o_ref[...] = x * jax.nn.sigmoid(x)


def _combine(g, m, wm):
    grid = (N_EDGES // EB,)
    return pl.pallas_call(
        _combine_body,
        grid=grid,
        in_specs=[
            pl.BlockSpec((EB, OUT_F), lambda i: (i, 0)),
            pl.BlockSpec((EB, EDGE_F), lambda i: (i, 0)),
            pl.BlockSpec((EDGE_F, OUT_F), lambda i: (0, 0)),
        ],
        out_specs=pl.BlockSpec((EB, OUT_F), lambda i: (i, 0)),
        out_shape=jax.ShapeDtypeStruct((N_EDGES, OUT_F), jnp.float32),
    )(g, m, wm)


@jax.jit
def kernel(h, m, edge_index, W):
    ws = W[:ATOM]
    wt = W[ATOM : 2 * ATOM]
    wm = W[2 * ATOM :]
    src = edge_index[0].astype(jnp.int32).reshape(NW, NCHUNK, CH)
    dst = edge_index[1].astype(jnp.int32).reshape(NW, NCHUNK, CH)

    a, b = _project(h, ws, wt)
    g = _gather_add(a, b, src, dst)
    return _combine(g, m, wm)
